# baseline (speedup 1.0000x reference)
import functools

import jax
import jax.numpy as jnp
from jax import lax
from jax.experimental import pallas as pl
from jax.experimental.pallas import tpu as pltpu
from jax.experimental.pallas import tpu_sc as plsc

_B = 2
_H = 16
_K = 2048
_Q = 2048
_NB = 32
_LINE = 4096
_JH = _LINE // 2


@functools.partial(
    pl.kernel,
    mesh=plsc.VectorSubcoreMesh(core_axis_name="c", subcore_axis_name="s"),
    out_type=jax.ShapeDtypeStruct((_H, 1, _LINE), jnp.float32),
    scratch_types=[
        pltpu.VMEM((_NB, _H), jnp.float32),
        pltpu.VMEM((_JH,), jnp.float32),
    ],
    compiler_params=pltpu.CompilerParams(needs_layout_passes=False),
)
def _sc_line(table_hbm, gline_hbm, table_v, chunk_v):
    h = lax.axis_index("s")
    base = lax.axis_index("c") * _JH
    pltpu.sync_copy(table_hbm, table_v)
    hvec = jnp.full((16,), h, jnp.int32)
    for g in range(_JH // 16):
        j = base + g * 16 + lax.iota(jnp.int32, 16)
        d = 2047 - j
        nb = _NB // 2
        rb = jnp.where(d > 0, nb, 0)
        ad = jnp.abs(d)
        max_exact = nb // 2
        sq = (ad * ad).astype(jnp.float32)
        e = (lax.bitcast_convert_type(sq, jnp.int32) >> 23) - 127
        if_large = jnp.minimum(max_exact + (e - 6), nb - 1)
        bucket = rb + jnp.where(ad < max_exact, ad, if_large)
        chunk_v[pl.ds(g * 16, 16)] = plsc.load_gather(table_v, [bucket, hvec])
    pltpu.sync_copy(chunk_v, gline_hbm.at[h, 0, pl.ds(base, _JH)])


def _bcast_kernel(gline_ref, out_ref):
    big = jnp.broadcast_to(gline_ref[0], (_K, _LINE))
    rolled = pltpu.roll(big, _LINE // 2 + 1, 1, stride=1, stride_axis=0)
    out_ref[0, 0] = rolled[:, :_Q]


def kernel(batch, key_length, query_length, table):
    gline = _sc_line(table)
    out = pl.pallas_call(
        _bcast_kernel,
        grid=(_H, _B),
        in_specs=[pl.BlockSpec((1, 1, _LINE), lambda h, b: (h, 0, 0))],
        out_specs=pl.BlockSpec((1, 1, _K, _Q), lambda h, b: (b, h, 0, 0)),
        out_shape=jax.ShapeDtypeStruct((_B, _H, _K, _Q), jnp.float32),
        compiler_params=pltpu.CompilerParams(vmem_limit_bytes=100 * 1024 * 1024),
    )(gline)
    return out


# speedup vs baseline: 1.0366x; 1.0366x over previous
import functools

import jax
import jax.numpy as jnp
from jax import lax
from jax.experimental import pallas as pl
from jax.experimental.pallas import tpu as pltpu
from jax.experimental.pallas import tpu_sc as plsc

_B = 2
_H = 16
_K = 2048
_Q = 2048
_NB = 32
_LINE = 4096
_JH = _LINE // 2


@functools.partial(
    pl.kernel,
    mesh=plsc.VectorSubcoreMesh(core_axis_name="c", subcore_axis_name="s"),
    out_type=jax.ShapeDtypeStruct((_H, 1, _LINE), jnp.float32),
    scratch_types=[
        pltpu.VMEM((_NB, _H), jnp.float32),
        pltpu.VMEM((_JH,), jnp.float32),
    ],
    compiler_params=pltpu.CompilerParams(needs_layout_passes=False),
)
def _sc_line(table_hbm, gline_hbm, table_v, chunk_v):
    h = lax.axis_index("s")
    base = lax.axis_index("c") * _JH
    pltpu.sync_copy(table_hbm, table_v)
    hvec = jnp.full((16,), h, jnp.int32)

    def body(g, carry):
        j = base + g * 16 + lax.iota(jnp.int32, 16)
        d = 2047 - j
        nb = _NB // 2
        rb = jnp.where(d > 0, nb, 0)
        ad = jnp.abs(d)
        max_exact = nb // 2
        sq = (ad * ad).astype(jnp.float32)
        e = (lax.bitcast_convert_type(sq, jnp.int32) >> 23) - 127
        if_large = jnp.minimum(max_exact + (e - 6), nb - 1)
        bucket = rb + jnp.where(ad < max_exact, ad, if_large)
        chunk_v[pl.ds(g * 16, 16)] = plsc.load_gather(table_v, [bucket, hvec])
        return carry

    lax.fori_loop(0, _JH // 16, body, 0)
    pltpu.sync_copy(chunk_v, gline_hbm.at[h, 0, pl.ds(base, _JH)])


def _bcast_kernel(gline_ref, out_ref):
    big = jnp.broadcast_to(gline_ref[0], (_K, _LINE))
    rolled = pltpu.roll(big, _LINE // 2 + 1, 1, stride=1, stride_axis=0)
    out_ref[0, 0] = rolled[:, :_Q]


def kernel(batch, key_length, query_length, table):
    gline = _sc_line(table)
    out = pl.pallas_call(
        _bcast_kernel,
        grid=(_H, _B),
        in_specs=[pl.BlockSpec((1, 1, _LINE), lambda h, b: (h, 0, 0))],
        out_specs=pl.BlockSpec((1, 1, _K, _Q), lambda h, b: (b, h, 0, 0)),
        out_shape=jax.ShapeDtypeStruct((_B, _H, _K, _Q), jnp.float32),
        compiler_params=pltpu.CompilerParams(vmem_limit_bytes=100 * 1024 * 1024),
    )(gline)
    return out
